# baseline (speedup 1.0000x reference)
import functools

import jax
import jax.numpy as jnp
from jax import lax
from jax.experimental import pallas as pl
from jax.experimental.pallas import tpu as pltpu
from jax.experimental.pallas import tpu_sc as plsc

NC = 2
NS = 16
NW = NC * NS
G = 8


def _emb_call(b, h, d):
    s_per_w = b // NW
    n_groups = s_per_w // G
    assert n_groups % 2 == 0
    mesh = plsc.VectorSubcoreMesh(
        core_axis_name="c", subcore_axis_name="s",
        num_cores=NC, num_subcores=NS)

    @functools.partial(
        pl.kernel,
        out_type=jax.ShapeDtypeStruct((b, h, d), jnp.float32),
        mesh=mesh,
        scratch_types=[
            pltpu.VMEM((s_per_w, h), jnp.int32),
            pltpu.VMEM((2, G, h, d), jnp.float32),
            pltpu.SemaphoreType.DMA,
            pltpu.SemaphoreType.DMA,
        ],
        compiler_params=pltpu.CompilerParams(use_tc_tiling_on_sc=False),
    )
    def emb(ids_hbm, table_hbm, out_hbm, idx_v, rows_v, sem0, sem1):
        wid = lax.axis_index("s") * NC + lax.axis_index("c")
        sent0 = wid * s_per_w
        pltpu.sync_copy(ids_hbm.at[pl.ds(sent0, s_per_w)], idx_v)

        bufs = (rows_v.at[0], rows_v.at[1])
        sems = (sem0, sem1)

        def fire(g, bf):
            for j in range(G):
                pltpu.async_copy(
                    table_hbm.at[idx_v.at[g * G + j]],
                    bufs[bf].at[j], sems[bf])

        def drain(g, bf):
            for j in range(G):
                pltpu.make_async_copy(
                    table_hbm.at[idx_v.at[g * G + j]],
                    bufs[bf].at[j], sems[bf]).wait()
            pltpu.sync_copy(bufs[bf], out_hbm.at[pl.ds(sent0 + g * G, G)])

        fire(0, 0)

        def step(i, carry):
            g = 2 * i
            fire(g + 1, 1)
            drain(g, 0)
            fire(g + 2, 0)
            drain(g + 1, 1)
            return carry

        lax.fori_loop(0, n_groups // 2 - 1, step, 0)
        g = n_groups - 2
        fire(g + 1, 1)
        drain(g, 0)
        drain(g + 1, 1)

    return emb


def kernel(token_ids, p_emb):
    b, h = token_ids.shape
    v, d = p_emb.shape
    return _emb_call(b, h, d)(token_ids.astype(jnp.int32), p_emb)
